# initial kernel scaffold (unmeasured)
import jax
import jax.numpy as jnp
from jax import lax
from jax.experimental import pallas as pl
from jax.experimental.pallas import tpu as pltpu

N_DEV = 4
M_PER = 1024
K = 4096
N_TOTAL = 8192
N_PER = N_TOTAL // N_DEV
KSUB = 512
NSUB = N_PER // KSUB
F32 = jnp.float32
BF16 = jnp.bfloat16


def kernel(x, w_mat):
    x = x.astype(BF16)

    def body(x_ref, w_hbm, out_hbm, w_vmem, stage, recv, amax_buf,
             w_sems, send_sems, recv_sems, a_send_sems, a_recv_sems,
             out_sems):
        me = lax.axis_index("i")

        barrier = pltpu.get_barrier_semaphore()
        for d in range(1, N_DEV):
            peer = lax.rem(me + d, N_DEV)
            pl.semaphore_signal(
                barrier, inc=1, device_id=(peer,),
                device_id_type=pl.DeviceIdType.MESH,
            )
        pl.semaphore_wait(barrier, N_DEV - 1)

        sched = [(d, k) for d in (1, 2, 3, 0) for k in range(NSUB)]

        def w_copy(i, buf):
            d, k = sched[i]
            j = lax.rem(me + d, N_DEV)
            col = j * N_PER + k * KSUB
            return pltpu.make_async_copy(
                w_hbm.at[:, pl.ds(col, KSUB)], w_vmem.at[buf],
                w_sems.at[buf],
            )

        data_rdmas = []
        amax = jnp.zeros((), F32)
        w_copy(0, 0).start()
        for i, (d, k) in enumerate(sched):
            buf = i % 2
            if i + 1 < len(sched):
                w_copy(i + 1, (i + 1) % 2).start()
            w_copy(i, buf).wait()
            acc = jnp.dot(
                x_ref[...], w_vmem[buf].astype(BF16),
                preferred_element_type=F32,
            )
            y = jnp.maximum(acc, 0.0)
            amax = jnp.maximum(amax, jnp.max(y))
            yb = y.astype(BF16)
            ksl = slice(k * KSUB, (k + 1) * KSUB)
            if d == 0:
                recv[0, :, ksl] = yb
            else:
                stage[d - 1, :, ksl] = yb
                if k == NSUB - 1:
                    e = N_DEV - d
                    peer = lax.rem(me + d, N_DEV)
                    rdma = pltpu.make_async_remote_copy(
                        src_ref=stage.at[d - 1],
                        dst_ref=recv.at[e],
                        send_sem=send_sems.at[d - 1],
                        recv_sem=recv_sems.at[e],
                        device_id=(peer,),
                        device_id_type=pl.DeviceIdType.MESH,
                    )
                    rdma.start()
                    data_rdmas.append(rdma)

        amax_buf[0, :, :] = jnp.full((8, 128), amax, F32)
        amax_rdmas = []
        for d in range(1, N_DEV):
            e = N_DEV - d
            peer = lax.rem(me + d, N_DEV)
            rdma = pltpu.make_async_remote_copy(
                src_ref=amax_buf.at[0],
                dst_ref=amax_buf.at[e],
                send_sem=a_send_sems.at[d - 1],
                recv_sem=a_recv_sems.at[e],
                device_id=(peer,),
                device_id_type=pl.DeviceIdType.MESH,
            )
            rdma.start()
            amax_rdmas.append(rdma)

        for e in range(1, N_DEV):
            pltpu.make_async_remote_copy(
                src_ref=amax_buf.at[0],
                dst_ref=amax_buf.at[e],
                send_sem=a_send_sems.at[0],
                recv_sem=a_recv_sems.at[e],
                device_id=(me,),
                device_id_type=pl.DeviceIdType.MESH,
            ).wait_recv()

        g = amax_buf[0, 0, 0]
        for e in range(1, N_DEV):
            g = jnp.maximum(g, amax_buf[e, 0, 0])
        scale = g / 448.0
        inv = 448.0 / g

        out_copies = []
        for e in range(N_DEV):
            if e > 0:
                pltpu.make_async_remote_copy(
                    src_ref=stage.at[0],
                    dst_ref=recv.at[e],
                    send_sem=send_sems.at[0],
                    recv_sem=recv_sems.at[e],
                    device_id=(me,),
                    device_id_type=pl.DeviceIdType.MESH,
                ).wait_recv()
            v = recv[e, :, :].astype(F32)
            t = jnp.minimum(v * inv, 448.0)
            q = t.astype(jnp.float8_e4m3fn)
            recv[e, :, :] = (q.astype(F32) * scale).astype(BF16)
            row = lax.rem(me + e, N_DEV) * M_PER
            cp = pltpu.make_async_copy(
                recv.at[e], out_hbm.at[pl.ds(row, M_PER)], out_sems.at[e],
            )
            cp.start()
            out_copies.append(cp)

        for cp in out_copies:
            cp.wait()
        for rdma in data_rdmas:
            rdma.wait_send()
        for rdma in amax_rdmas:
            rdma.wait_send()

    return pl.pallas_call(
        body,
        out_shape=jax.ShapeDtypeStruct((N_DEV * M_PER, N_PER), BF16),
        in_specs=[
            pl.BlockSpec(memory_space=pltpu.VMEM),
            pl.BlockSpec(memory_space=pl.ANY),
        ],
        out_specs=pl.BlockSpec(memory_space=pl.ANY),
        scratch_shapes=[
            pltpu.VMEM((2, K, KSUB), F32),
            pltpu.VMEM((N_DEV - 1, M_PER, N_PER), BF16),
            pltpu.VMEM((N_DEV, M_PER, N_PER), BF16),
            pltpu.VMEM((N_DEV, 8, 128), F32),
            pltpu.SemaphoreType.DMA((2,)),
            pltpu.SemaphoreType.DMA((N_DEV - 1,)),
            pltpu.SemaphoreType.DMA((N_DEV,)),
            pltpu.SemaphoreType.DMA((N_DEV - 1,)),
            pltpu.SemaphoreType.DMA((N_DEV,)),
            pltpu.SemaphoreType.DMA((N_DEV,)),
        ],
        compiler_params=pltpu.CompilerParams(collective_id=0),
    )(x, w_mat)


# baseline (device time: 175380 ns/iter reference)
import jax
import jax.numpy as jnp
from jax import lax
from jax.experimental import pallas as pl
from jax.experimental.pallas import tpu as pltpu

N_DEV = 4
M_PER = 1024
K = 4096
N_TOTAL = 8192
N_PER = N_TOTAL // N_DEV
KSUB = 512
NSUB = N_PER // KSUB
F32 = jnp.float32
BF16 = jnp.bfloat16


def kernel(x, w_mat):
    x = x.astype(BF16)

    def body(x_ref, w_hbm, out_hbm, w_vmem, stage, recv, amax_buf,
             w_sems, send_sems, recv_sems, a_send_sems, a_recv_sems,
             out_sems):
        me = lax.axis_index("i")

        barrier = pltpu.get_barrier_semaphore()
        for d in range(1, N_DEV):
            peer = lax.rem(me + d, N_DEV)
            pl.semaphore_signal(
                barrier, inc=1, device_id=(peer,),
                device_id_type=pl.DeviceIdType.MESH,
            )
        pl.semaphore_wait(barrier, N_DEV - 1)

        sched = [(d, k) for d in (1, 2, 3, 0) for k in range(NSUB)]

        def w_copy(i, buf):
            d, k = sched[i]
            j = lax.rem(me + d, N_DEV)
            col = j * N_PER + k * KSUB
            return pltpu.make_async_copy(
                w_hbm.at[:, pl.ds(col, KSUB)], w_vmem.at[buf],
                w_sems.at[buf],
            )

        data_rdmas = []
        amax = jnp.zeros((), F32)
        w_copy(0, 0).start()
        for i, (d, k) in enumerate(sched):
            buf = i % 2
            if i + 1 < len(sched):
                w_copy(i + 1, (i + 1) % 2).start()
            w_copy(i, buf).wait()
            acc = jnp.dot(
                x_ref[...], w_vmem[buf].astype(BF16),
                preferred_element_type=F32,
            )
            y = jnp.maximum(acc, 0.0)
            amax = jnp.maximum(amax, jnp.max(y))
            yb = y.astype(BF16)
            ksl = slice(k * KSUB, (k + 1) * KSUB)
            if d == 0:
                recv[0, :, ksl] = yb
            else:
                stage[d - 1, :, ksl] = yb
                if k == NSUB - 1:
                    e = N_DEV - d
                    peer = lax.rem(me + d, N_DEV)
                    rdma = pltpu.make_async_remote_copy(
                        src_ref=stage.at[d - 1],
                        dst_ref=recv.at[e],
                        send_sem=send_sems.at[d - 1],
                        recv_sem=recv_sems.at[e],
                        device_id=(peer,),
                        device_id_type=pl.DeviceIdType.MESH,
                    )
                    rdma.start()
                    data_rdmas.append(rdma)

        amax_buf[0, :, :] = jnp.full((8, 128), amax, F32)
        amax_rdmas = []
        for d in range(1, N_DEV):
            e = N_DEV - d
            peer = lax.rem(me + d, N_DEV)
            rdma = pltpu.make_async_remote_copy(
                src_ref=amax_buf.at[0],
                dst_ref=amax_buf.at[e],
                send_sem=a_send_sems.at[d - 1],
                recv_sem=a_recv_sems.at[e],
                device_id=(peer,),
                device_id_type=pl.DeviceIdType.MESH,
            )
            rdma.start()
            amax_rdmas.append(rdma)

        for e in range(1, N_DEV):
            pltpu.make_async_remote_copy(
                src_ref=amax_buf.at[0],
                dst_ref=amax_buf.at[e],
                send_sem=a_send_sems.at[0],
                recv_sem=a_recv_sems.at[e],
                device_id=(me,),
                device_id_type=pl.DeviceIdType.MESH,
            ).wait_recv()

        g = amax_buf[0, 0, 0]
        for e in range(1, N_DEV):
            g = jnp.maximum(g, amax_buf[e, 0, 0])
        scale = g / 448.0
        inv = 448.0 / g

        out_copies = []
        for e in range(N_DEV):
            if e > 0:
                pltpu.make_async_remote_copy(
                    src_ref=stage.at[0],
                    dst_ref=recv.at[e],
                    send_sem=send_sems.at[0],
                    recv_sem=recv_sems.at[e],
                    device_id=(me,),
                    device_id_type=pl.DeviceIdType.MESH,
                ).wait_recv()
            v = recv[e, :, :].astype(F32)
            t = jnp.minimum(v * inv, 448.0)
            q = t.astype(jnp.float8_e4m3fn)
            recv[e, :, :] = (q.astype(F32) * scale).astype(BF16)
            row = lax.rem(me + e, N_DEV) * M_PER
            cp = pltpu.make_async_copy(
                recv.at[e], out_hbm.at[pl.ds(row, M_PER)], out_sems.at[e],
            )
            cp.start()
            out_copies.append(cp)

        for cp in out_copies:
            cp.wait()
        for rdma in data_rdmas:
            rdma.wait_send()
        for rdma in amax_rdmas:
            rdma.wait_send()

    return pl.pallas_call(
        body,
        out_shape=jax.ShapeDtypeStruct((N_DEV * M_PER, N_PER), BF16),
        in_specs=[
            pl.BlockSpec(memory_space=pltpu.VMEM),
            pl.BlockSpec(memory_space=pl.ANY),
        ],
        out_specs=pl.BlockSpec(memory_space=pl.ANY),
        scratch_shapes=[
            pltpu.VMEM((2, K, KSUB), F32),
            pltpu.VMEM((N_DEV - 1, M_PER, N_PER), BF16),
            pltpu.VMEM((N_DEV, M_PER, N_PER), BF16),
            pltpu.VMEM((N_DEV, 8, 128), F32),
            pltpu.SemaphoreType.DMA((2,)),
            pltpu.SemaphoreType.DMA((N_DEV - 1,)),
            pltpu.SemaphoreType.DMA((N_DEV,)),
            pltpu.SemaphoreType.DMA((N_DEV - 1,)),
            pltpu.SemaphoreType.DMA((N_DEV,)),
            pltpu.SemaphoreType.DMA((N_DEV,)),
        ],
        compiler_params=pltpu.CompilerParams(
            collective_id=0, vmem_limit_bytes=64 * 1024 * 1024,
        ),
    )(x, w_mat)


# device time: 149147 ns/iter; 1.1759x vs baseline; 1.1759x over previous
import jax
import jax.numpy as jnp
from jax import lax
from jax.experimental import pallas as pl
from jax.experimental.pallas import tpu as pltpu

N_DEV = 4
M_PER = 1024
K = 4096
N_TOTAL = 8192
N_PER = N_TOTAL // N_DEV
KSUB = 512
NSUB = N_PER // KSUB
F32 = jnp.float32
BF16 = jnp.bfloat16


def kernel(x, w_mat):
    x = x.astype(BF16)

    def body(x_ref, w_hbm, out_hbm, w_vmem, stage, recv, amax_buf,
             fp8_send, fp8_recv,
             w_sems, send_sems, recv_sems, a_send_sems, a_recv_sems,
             out_sems, diag_sems):
        me = lax.axis_index("i")

        barrier = pltpu.get_barrier_semaphore()
        for d in range(1, N_DEV):
            peer = lax.rem(me + d, N_DEV)
            pl.semaphore_signal(
                barrier, inc=1, device_id=(peer,),
                device_id_type=pl.DeviceIdType.MESH,
            )
        pl.semaphore_wait(barrier, N_DEV - 1)

        sched = [(d, k) for d in (1, 3, 2, 0) for k in range(NSUB)]

        def w_copy(i, buf):
            d, k = sched[i]
            j = lax.rem(me + d, N_DEV)
            col = j * N_PER + k * KSUB
            return pltpu.make_async_copy(
                w_hbm.at[:, pl.ds(col, KSUB)], w_vmem.at[buf],
                w_sems.at[buf],
            )

        data_rdmas = []
        amax = jnp.zeros((), F32)
        w_copy(0, 0).start()
        for i, (d, k) in enumerate(sched):
            buf = i % 2
            if i + 1 < len(sched):
                w_copy(i + 1, (i + 1) % 2).start()
            w_copy(i, buf).wait()
            acc = jnp.dot(
                x_ref[...], w_vmem[buf].astype(BF16),
                preferred_element_type=F32,
            )
            y = jnp.maximum(acc, 0.0)
            amax = jnp.maximum(amax, jnp.max(y))
            yb = y.astype(BF16)
            ksl = slice(k * KSUB, (k + 1) * KSUB)
            if d == 0:
                recv[0, :, ksl] = yb
            else:
                stage[d - 1, :, ksl] = yb
                if k == NSUB - 1 and d != 2:
                    e = N_DEV - d
                    peer = lax.rem(me + d, N_DEV)
                    rdma = pltpu.make_async_remote_copy(
                        src_ref=stage.at[d - 1],
                        dst_ref=recv.at[e],
                        send_sem=send_sems.at[d - 1],
                        recv_sem=recv_sems.at[e],
                        device_id=(peer,),
                        device_id_type=pl.DeviceIdType.MESH,
                    )
                    rdma.start()
                    data_rdmas.append(rdma)

        amax_buf[0, :, :] = jnp.full((8, 128), amax, F32)
        amax_rdmas = []
        for d in range(1, N_DEV):
            e = N_DEV - d
            peer = lax.rem(me + d, N_DEV)
            rdma = pltpu.make_async_remote_copy(
                src_ref=amax_buf.at[0],
                dst_ref=amax_buf.at[e],
                send_sem=a_send_sems.at[d - 1],
                recv_sem=a_recv_sems.at[e],
                device_id=(peer,),
                device_id_type=pl.DeviceIdType.MESH,
            )
            rdma.start()
            amax_rdmas.append(rdma)

        for e in range(1, N_DEV):
            pltpu.make_async_remote_copy(
                src_ref=amax_buf.at[0],
                dst_ref=amax_buf.at[e],
                send_sem=a_send_sems.at[0],
                recv_sem=a_recv_sems.at[e],
                device_id=(me,),
                device_id_type=pl.DeviceIdType.MESH,
            ).wait_recv()

        g = amax_buf[0, 0, 0]
        for e in range(1, N_DEV):
            g = jnp.maximum(g, amax_buf[e, 0, 0])
        scale = g / 448.0
        inv = 448.0 / g

        v = stage[1, :, :].astype(F32)
        fp8_send[...] = jnp.minimum(v * inv, 448.0).astype(jnp.float8_e4m3fn)
        diag_peer = lax.rem(me + 2, N_DEV)
        diag_rdma = pltpu.make_async_remote_copy(
            src_ref=fp8_send,
            dst_ref=fp8_recv,
            send_sem=diag_sems.at[0],
            recv_sem=diag_sems.at[1],
            device_id=(diag_peer,),
            device_id_type=pl.DeviceIdType.MESH,
        )
        diag_rdma.start()

        out_copies = []
        for e in (0, 1, 3):
            if e > 0:
                pltpu.make_async_remote_copy(
                    src_ref=stage.at[0],
                    dst_ref=recv.at[e],
                    send_sem=send_sems.at[0],
                    recv_sem=recv_sems.at[e],
                    device_id=(me,),
                    device_id_type=pl.DeviceIdType.MESH,
                ).wait_recv()
            v = recv[e, :, :].astype(F32)
            t = jnp.minimum(v * inv, 448.0)
            q = t.astype(jnp.float8_e4m3fn)
            recv[e, :, :] = (q.astype(F32) * scale).astype(BF16)
            row = lax.rem(me + e, N_DEV) * M_PER
            cp = pltpu.make_async_copy(
                recv.at[e], out_hbm.at[pl.ds(row, M_PER)], out_sems.at[e],
            )
            cp.start()
            out_copies.append(cp)

        diag_rdma.wait_recv()
        recv[2, :, :] = (fp8_recv[...].astype(F32) * scale).astype(BF16)
        row = lax.rem(me + 2, N_DEV) * M_PER
        cp = pltpu.make_async_copy(
            recv.at[2], out_hbm.at[pl.ds(row, M_PER)], out_sems.at[2],
        )
        cp.start()
        out_copies.append(cp)

        for cp in out_copies:
            cp.wait()
        for rdma in data_rdmas:
            rdma.wait_send()
        for rdma in amax_rdmas:
            rdma.wait_send()
        diag_rdma.wait_send()

    return pl.pallas_call(
        body,
        out_shape=jax.ShapeDtypeStruct((N_DEV * M_PER, N_PER), BF16),
        in_specs=[
            pl.BlockSpec(memory_space=pltpu.VMEM),
            pl.BlockSpec(memory_space=pl.ANY),
        ],
        out_specs=pl.BlockSpec(memory_space=pl.ANY),
        scratch_shapes=[
            pltpu.VMEM((2, K, KSUB), F32),
            pltpu.VMEM((N_DEV - 1, M_PER, N_PER), BF16),
            pltpu.VMEM((N_DEV, M_PER, N_PER), BF16),
            pltpu.VMEM((N_DEV, 8, 128), F32),
            pltpu.VMEM((M_PER, N_PER), jnp.float8_e4m3fn),
            pltpu.VMEM((M_PER, N_PER), jnp.float8_e4m3fn),
            pltpu.SemaphoreType.DMA((2,)),
            pltpu.SemaphoreType.DMA((N_DEV - 1,)),
            pltpu.SemaphoreType.DMA((N_DEV,)),
            pltpu.SemaphoreType.DMA((N_DEV - 1,)),
            pltpu.SemaphoreType.DMA((N_DEV,)),
            pltpu.SemaphoreType.DMA((N_DEV,)),
            pltpu.SemaphoreType.DMA((2,)),
        ],
        compiler_params=pltpu.CompilerParams(
            collective_id=0, vmem_limit_bytes=64 * 1024 * 1024,
        ),
    )(x, w_mat)
